# baseline (speedup 1.0000x reference)
import jax
import jax.numpy as jnp
from jax.experimental import pallas as pl
from jax.experimental.pallas import tpu as pltpu

N = 5000
NPAD = 5120
BLK = 128
NB = NPAD // BLK
NCLS = 80


def _prep_body(ft_ref, at_ref, isz_ref, sc_ref, lab_ref, l_ref, t_ref, r_ref, b_ref):
    f = ft_ref[0]
    H = isz_ref[0, 0, 0].astype(jnp.float32)
    W = isz_ref[0, 0, 1].astype(jnp.float32)
    cls = f[5:5 + NCLS, :]
    mx = jnp.max(cls, axis=0, keepdims=True)
    rowio = jax.lax.broadcasted_iota(jnp.int32, (NCLS, NPAD), 0)
    lab = jnp.min(jnp.where(cls == mx, rowio, jnp.int32(2 ** 30)), axis=0, keepdims=True)
    sc = mx * f[4:5, :]

    cell_x = at_ref[0:1, :]
    cell_y = at_ref[1:2, :]
    stride = at_ref[2:3, :]
    w_a = at_ref[3:4, :]
    h_a = at_ref[4:5, :]
    cx = (f[0:1, :] + cell_x) * stride
    cy = (f[1:2, :] + cell_y) * stride
    w = w_a * jnp.exp(f[2:3, :])
    h = h_a * jnp.exp(f[3:4, :])
    l_un = cx - w / 2.0
    t_un = cy - h / 2.0
    r_un = l_un + w
    b_un = t_un + h
    l = jnp.clip(l_un, 0.0, W - 1.0)
    r = jnp.clip(r_un, 0.0, W - 1.0)
    t = jnp.clip(t_un, 0.0, H - 1.0)
    b = jnp.clip(b_un, 0.0, H - 1.0)

    sc_ref[0] = sc
    lab_ref[0] = lab
    l_ref[0] = l
    t_ref[0] = t
    r_ref[0] = r
    b_ref[0] = b


def _nms_body(scr_ref, lr_ref, tr_ref, rr_ref, br_ref, labf_ref,
              isz_ref, iszo_ref, st_ref, nt_ref,
              kl_ref, kt_ref, kr_ref, kb_ref, ksc_ref, keep_ref,
              origC, gsrc, ohT, sortC, sortRows, rrowS, labMM, iouS, auxS):
    thr = st_ref[0, 0]
    nmsT = nt_ref[0, 0]
    H = isz_ref[0, 0, 0].astype(jnp.float32)
    scale = iszo_ref[0, 0, 0].astype(jnp.float32) / H

    sc = scr_ref[0]
    maxc = jnp.max(jnp.maximum(rr_ref[0], br_ref[0])) + 1.0
    nvalid = jnp.sum((sc >= thr).astype(jnp.int32))
    nb_used = (nvalid + (BLK - 1)) // BLK

    def _split3(x):
        hi = x.astype(jnp.bfloat16)
        r1 = x - hi.astype(jnp.float32)
        mid = r1.astype(jnp.bfloat16)
        r2 = r1 - mid.astype(jnp.float32)
        return [hi, mid, r2.astype(jnp.bfloat16)]

    def _build_cols(o, _):
        lab_row = labf_ref[0, pl.dslice(o, 1), :]
        off_row = lab_row * maxc
        sc_row = scr_ref[0, pl.dslice(o, 1), :]
        parts = (_split3(lr_ref[0, pl.dslice(o, 1), :] + off_row)
                 + _split3(tr_ref[0, pl.dslice(o, 1), :] + off_row)
                 + _split3(rr_ref[0, pl.dslice(o, 1), :] + off_row)
                 + _split3(br_ref[0, pl.dslice(o, 1), :] + off_row))
        stack16 = jnp.concatenate(
            parts + [(sc_row >= thr).astype(jnp.bfloat16),
                     lab_row.astype(jnp.bfloat16),
                     jnp.zeros((2, BLK), jnp.bfloat16)], axis=0)
        gsrc[pl.dslice(o * BLK, BLK), :] = jnp.transpose(stack16)
        stack = jnp.concatenate([sc_row, jnp.zeros((1, BLK), jnp.float32),
                                 lab_row, (sc_row >= thr).astype(jnp.float32),
                                 jnp.zeros((4, BLK), jnp.float32)], axis=0)
        origC[pl.dslice(o * BLK, BLK), :] = jnp.transpose(stack)
        return 0

    jax.lax.fori_loop(0, NB, _build_cols, 0, unroll=False)

    tri = (jax.lax.broadcasted_iota(jnp.int32, (BLK, BLK), 1)
           < jax.lax.broadcasted_iota(jnp.int32, (BLK, BLK), 0))

    rrowS[:, :] = jnp.zeros((NB, BLK), jnp.float32)

    def _beats(s_j, lab_j, v_j, s_i, lab_i, v_i):
        return (v_j > v_i) | ((v_j == v_i) & (
            (lab_j < lab_i) | ((lab_j == lab_i) & (s_j > s_i))))

    def _rank_blk(a, _):
        s_i = origC[pl.dslice(a * BLK, BLK), 0:1]
        lab_i = origC[pl.dslice(a * BLK, BLK), 2:3]
        v_i = origC[pl.dslice(a * BLK, BLK), 3:4]

        def _pair(bb, acc):
            s_j = scr_ref[0, pl.dslice(bb, 1), :]
            lab_j = labf_ref[0, pl.dslice(bb, 1), :]
            v_j = (s_j >= thr).astype(jnp.float32)
            T = _beats(s_j, lab_j, v_j, s_i, lab_i, v_i).astype(jnp.float32)
            colsum = jnp.sum(T, axis=0, keepdims=True)
            rrowS[pl.dslice(bb, 1), :] = rrowS[pl.dslice(bb, 1), :] + (128.0 - colsum)
            return acc + T

        acc = jax.lax.fori_loop(a + 1, NB, _pair,
                                jnp.zeros((BLK, BLK), jnp.float32))
        s_a = scr_ref[0, pl.dslice(a, 1), :]
        lab_a = labf_ref[0, pl.dslice(a, 1), :]
        v_a = (s_a >= thr).astype(jnp.float32)
        full_tie = (v_a == v_i) & (lab_a == lab_i) & (s_a == s_i)
        acc = acc + (_beats(s_a, lab_a, v_a, s_i, lab_i, v_i)
                     | (full_tie & tri)).astype(jnp.float32)
        total = jnp.sum(acc, axis=1, keepdims=True) + \
            jnp.transpose(rrowS[pl.dslice(a, 1), :])
        origC[pl.dslice(a * BLK, BLK), 1:2] = total
        return 0

    jax.lax.fori_loop(0, NB, _rank_blk, 0, unroll=False)

    lane_f = jax.lax.broadcasted_iota(jnp.int32, (1, BLK), 1).astype(jnp.float32)

    def _gather_blk(rr, _):
        tgt = (rr * BLK).astype(jnp.float32) + lane_f

        def _oh(o, _2):
            rank_col = origC[pl.dslice(o * BLK, BLK), 1:2]
            ohT[pl.dslice(o * BLK, BLK), :] = (rank_col == tgt).astype(jnp.bfloat16)
            return 0

        jax.lax.fori_loop(0, NB, _oh, 0, unroll=8)
        rows16 = jax.lax.dot_general(gsrc[:, :], ohT[:, :],
                                     (((0,), (0,)), ((), ())),
                                     preferred_element_type=jnp.float32)
        lo_row = rows16[0:1, :] + rows16[1:2, :] + rows16[2:3, :]
        to_row = rows16[3:4, :] + rows16[4:5, :] + rows16[5:6, :]
        ro_row = rows16[6:7, :] + rows16[7:8, :] + rows16[8:9, :]
        bo_row = rows16[9:10, :] + rows16[10:11, :] + rows16[11:12, :]
        v_row = rows16[12:13, :]
        lab_row = rows16[13:14, :]
        area_row = jnp.maximum(ro_row - lo_row, 0.0) * jnp.maximum(bo_row - to_row, 0.0)
        rows8 = jnp.concatenate([lo_row, to_row, ro_row, bo_row, v_row,
                                 jnp.zeros((1, BLK), jnp.float32), area_row,
                                 jnp.zeros((1, BLK), jnp.float32)], axis=0)
        sortRows[pl.dslice(rr * 8, 8), :] = rows8
        sortC[pl.dslice(rr * BLK, BLK), :] = jnp.transpose(rows8)
        labMM[pl.dslice(rr, 1), 0:1] = jnp.max(
            jnp.where(v_row > 0.5, lab_row, -1.0), axis=1, keepdims=True)
        labMM[pl.dslice(rr, 1), 1:2] = jnp.min(
            jnp.where(v_row > 0.5, lab_row, 1e9), axis=1, keepdims=True)
        return 0

    jax.lax.fori_loop(0, nb_used, _gather_blk, 0, unroll=False)

    lane = jax.lax.broadcasted_iota(jnp.int32, (1, BLK), 1)

    def _nms_blk(q, _):
        cols = sortC[pl.dslice(q * BLK, BLK), :]
        l_i = cols[:, 0:1]
        t_i = cols[:, 1:2]
        r_i = cols[:, 2:3]
        b_i = cols[:, 3:4]
        v_i = cols[:, 4:5]
        area_i = cols[:, 6:7]

        def _iou_rows(rows8):
            l_j = rows8[0:1, :]
            t_j = rows8[1:2, :]
            r_j = rows8[2:3, :]
            b_j = rows8[3:4, :]
            area_j = rows8[6:7, :]
            iw = jnp.maximum(jnp.minimum(r_i, r_j) - jnp.maximum(l_i, l_j), 0.0)
            ih = jnp.maximum(jnp.minimum(b_i, b_j) - jnp.maximum(t_i, t_j), 0.0)
            inter = iw * ih
            union = area_i + area_j - inter
            return inter / jnp.maximum(union, 1e-9)

        def _prev(p, acc):
            rows8 = sortRows[pl.dslice(p * 8, 8), :]
            iou = _iou_rows(rows8)
            keep_j = rows8[5:6, :]
            return jnp.maximum(acc, jnp.where((keep_j > 0.5) & (iou > nmsT), 1.0, 0.0))

        mm = labMM[:, 0:1]
        minq = labMM[pl.dslice(q, 1), 1:2]
        pio = jax.lax.broadcasted_iota(jnp.int32, (NB, 1), 0)
        p0 = jnp.sum(jnp.where((mm < minq) & (pio < q), 1, 0)).astype(jnp.int32)
        p_start = jnp.where(nmsT > 0.0, p0, 0)

        sup_acc = jax.lax.fori_loop(p_start, q, _prev,
                                    jnp.zeros((BLK, BLK), jnp.float32))
        sup = jnp.max(sup_acc, axis=1, keepdims=True)

        iou_in = _iou_rows(sortRows[pl.dslice(q * 8, 8), :])
        v_j = jnp.transpose(v_i)
        conflict = jnp.max(jnp.where(tri & (iou_in > nmsT) & (v_i > 0.5) & (v_j > 0.5),
                                     1.0, 0.0)) > 0.5
        fast = v_i * (1.0 - sup)
        sortRows[pl.dslice(q * 8 + 5, 1), :] = jnp.transpose(fast)

        @pl.when(conflict)
        def _slow():
            iouS[:, :] = iou_in
            auxS[:, 0:1] = sup
            auxS[:, 1:2] = v_i

            def _seq(k, kb):
                row = iouS[pl.dslice(k, 1), :]
                sup_k = auxS[pl.dslice(k, 1), 0:1]
                val_k = auxS[pl.dslice(k, 1), 1:2]
                inblk = jnp.max(jnp.where((lane < k) & (kb > 0.5) & (row > nmsT), 1.0, 0.0),
                                axis=1, keepdims=True)
                kept = val_k * (1.0 - jnp.maximum(sup_k, inblk))
                return jnp.where(lane == k, kept, kb)

            kb = jax.lax.fori_loop(0, BLK, _seq, jnp.zeros((1, BLK), jnp.float32))
            sortRows[pl.dslice(q * 8 + 5, 1), :] = kb
        return 0

    jax.lax.fori_loop(0, nb_used, _nms_blk, 0, unroll=False)

    def _unperm(o, _):
        rank_col = origC[pl.dslice(o * BLK, BLK), 1:2]

        def _scan(s, acc):
            keep_row = sortRows[pl.dslice(s * 8 + 5, 1), :]
            srcpos = (s * BLK).astype(jnp.float32) + lane_f
            return jnp.maximum(acc, jnp.where((rank_col == srcpos) & (keep_row > 0.5),
                                              1.0, 0.0))

        acc = jax.lax.fori_loop(0, nb_used, _scan, jnp.zeros((BLK, BLK), jnp.float32))
        krow = jnp.transpose(jnp.max(acc, axis=1, keepdims=True))
        kl_ref[0, pl.dslice(o, 1), :] = lr_ref[0, pl.dslice(o, 1), :] * scale * krow
        kt_ref[0, pl.dslice(o, 1), :] = tr_ref[0, pl.dslice(o, 1), :] * scale * krow
        kr_ref[0, pl.dslice(o, 1), :] = rr_ref[0, pl.dslice(o, 1), :] * scale * krow
        kb_ref[0, pl.dslice(o, 1), :] = br_ref[0, pl.dslice(o, 1), :] * scale * krow
        ksc_ref[0, pl.dslice(o, 1), :] = scr_ref[0, pl.dslice(o, 1), :] * krow
        keep_ref[0, pl.dslice(o, 1), :] = krow
        return 0

    jax.lax.fori_loop(0, NB, _unperm, 0, unroll=False)


def kernel(features, anchors, image_sizes, image_sizes_ori, score_thresh, nms_thresh):
    B = features.shape[0]
    ft = jnp.transpose(features, (0, 2, 1))
    ft = jnp.pad(ft, ((0, 0), (0, 3), (0, NPAD - N)))
    at = jnp.pad(jnp.transpose(anchors, (1, 0)), ((0, 3), (0, NPAD - N)))
    st = jnp.reshape(jnp.asarray(score_thresh, jnp.float32), (1, 1))
    nt = jnp.reshape(jnp.asarray(nms_thresh, jnp.float32), (1, 1))

    f32 = jnp.float32
    sc, lab, l, t, r, b = pl.pallas_call(
        _prep_body,
        grid=(B,),
        in_specs=[
            pl.BlockSpec((1, 88, NPAD), lambda i: (i, 0, 0)),
            pl.BlockSpec((8, NPAD), lambda i: (0, 0)),
            pl.BlockSpec((1, 1, 2), lambda i: (i, 0, 0), memory_space=pltpu.SMEM),
        ],
        out_specs=[pl.BlockSpec((1, 1, NPAD), lambda i: (i, 0, 0))] * 6,
        out_shape=[
            jax.ShapeDtypeStruct((B, 1, NPAD), f32),
            jax.ShapeDtypeStruct((B, 1, NPAD), jnp.int32),
            jax.ShapeDtypeStruct((B, 1, NPAD), f32),
            jax.ShapeDtypeStruct((B, 1, NPAD), f32),
            jax.ShapeDtypeStruct((B, 1, NPAD), f32),
            jax.ShapeDtypeStruct((B, 1, NPAD), f32),
        ],
    )(ft, at, image_sizes.reshape(B, 1, 2))

    scr = sc.reshape(B, NB, BLK)
    lr = l.reshape(B, NB, BLK)
    tr = t.reshape(B, NB, BLK)
    rr = r.reshape(B, NB, BLK)
    br = b.reshape(B, NB, BLK)
    labf = lab.astype(f32).reshape(B, NB, BLK)

    row_spec = pl.BlockSpec((1, NB, BLK), lambda i: (i, 0, 0))
    smem2 = pl.BlockSpec((1, 1, 2), lambda i: (i, 0, 0), memory_space=pltpu.SMEM)
    smem1 = pl.BlockSpec((1, 1), lambda i: (0, 0), memory_space=pltpu.SMEM)
    outs = pl.pallas_call(
        _nms_body,
        grid=(B,),
        in_specs=[row_spec] * 6 + [smem2, smem2, smem1, smem1],
        out_specs=[row_spec] * 6,
        out_shape=[jax.ShapeDtypeStruct((B, NB, BLK), f32)] * 6,
        scratch_shapes=[
            pltpu.VMEM((NPAD, 8), f32),
            pltpu.VMEM((NPAD, 16), jnp.bfloat16),
            pltpu.VMEM((NPAD, BLK), jnp.bfloat16),
            pltpu.VMEM((NPAD, 8), f32),
            pltpu.VMEM((NB * 8, BLK), f32),
            pltpu.VMEM((NB, BLK), f32),
            pltpu.VMEM((NB, 8), f32),
            pltpu.VMEM((BLK, BLK), f32),
            pltpu.VMEM((BLK, 8), f32),
        ],
    )(scr, lr, tr, rr, br, labf, image_sizes.reshape(B, 1, 2),
      image_sizes_ori.reshape(B, 1, 2), st, nt)
    kl, kt, kr, kb, ksc, keepf = outs

    out_boxes = jnp.stack([
        kl.reshape(B, NPAD)[:, :N], kt.reshape(B, NPAD)[:, :N],
        kr.reshape(B, NPAD)[:, :N], kb.reshape(B, NPAD)[:, :N]], axis=-1)
    out_scores = ksc.reshape(B, NPAD)[:, :N]
    out_labels = lab.reshape(B, NPAD)[:, :N]
    out_keep = keepf.reshape(B, NPAD)[:, :N] > 0.5
    return (out_boxes, out_scores, out_labels, out_keep)


# speedup vs baseline: 1.1001x; 1.1001x over previous
import jax
import jax.numpy as jnp
from jax.experimental import pallas as pl
from jax.experimental.pallas import tpu as pltpu

N = 5000
NPAD = 5120
BLK = 128
NB = NPAD // BLK
NCLS = 80


def _prep_body(ft_ref, at_ref, isz_ref, sc_ref, lab_ref, l_ref, t_ref, r_ref, b_ref):
    f = ft_ref[0]
    H = isz_ref[0, 0, 0].astype(jnp.float32)
    W = isz_ref[0, 0, 1].astype(jnp.float32)
    cls = f[5:5 + NCLS, :]
    mx = jnp.max(cls, axis=0, keepdims=True)
    rowio = jax.lax.broadcasted_iota(jnp.int32, (NCLS, NPAD), 0)
    lab = jnp.min(jnp.where(cls == mx, rowio, jnp.int32(2 ** 30)), axis=0, keepdims=True)
    sc = mx * f[4:5, :]

    cell_x = at_ref[0:1, :]
    cell_y = at_ref[1:2, :]
    stride = at_ref[2:3, :]
    w_a = at_ref[3:4, :]
    h_a = at_ref[4:5, :]
    cx = (f[0:1, :] + cell_x) * stride
    cy = (f[1:2, :] + cell_y) * stride
    w = w_a * jnp.exp(f[2:3, :])
    h = h_a * jnp.exp(f[3:4, :])
    l_un = cx - w / 2.0
    t_un = cy - h / 2.0
    r_un = l_un + w
    b_un = t_un + h
    l = jnp.clip(l_un, 0.0, W - 1.0)
    r = jnp.clip(r_un, 0.0, W - 1.0)
    t = jnp.clip(t_un, 0.0, H - 1.0)
    b = jnp.clip(b_un, 0.0, H - 1.0)

    sc_ref[0] = sc
    lab_ref[0] = lab
    l_ref[0] = l
    t_ref[0] = t
    r_ref[0] = r
    b_ref[0] = b


def _nms_body(scr_ref, lr_ref, tr_ref, rr_ref, br_ref, labf_ref,
              isz_ref, iszo_ref, st_ref, nt_ref,
              kl_ref, kt_ref, kr_ref, kb_ref, ksc_ref, keep_ref,
              origC, gsrc, ohT, sortC, sortRows, rrowS, comboS, labMM, iouS, auxS):
    thr = st_ref[0, 0]
    nmsT = nt_ref[0, 0]
    H = isz_ref[0, 0, 0].astype(jnp.float32)
    scale = iszo_ref[0, 0, 0].astype(jnp.float32) / H

    sc = scr_ref[0]
    maxc = jnp.max(jnp.maximum(rr_ref[0], br_ref[0])) + 1.0
    nvalid = jnp.sum((sc >= thr).astype(jnp.int32))
    nb_used = (nvalid + (BLK - 1)) // BLK

    def _split3(x):
        hi = x.astype(jnp.bfloat16)
        r1 = x - hi.astype(jnp.float32)
        mid = r1.astype(jnp.bfloat16)
        r2 = r1 - mid.astype(jnp.float32)
        return [hi, mid, r2.astype(jnp.bfloat16)]

    def _build_cols(o, _):
        lab_row = labf_ref[0, pl.dslice(o, 1), :]
        off_row = lab_row * maxc
        sc_row = scr_ref[0, pl.dslice(o, 1), :]
        parts = (_split3(lr_ref[0, pl.dslice(o, 1), :] + off_row)
                 + _split3(tr_ref[0, pl.dslice(o, 1), :] + off_row)
                 + _split3(rr_ref[0, pl.dslice(o, 1), :] + off_row)
                 + _split3(br_ref[0, pl.dslice(o, 1), :] + off_row))
        stack16 = jnp.concatenate(
            parts + [(sc_row >= thr).astype(jnp.bfloat16),
                     lab_row.astype(jnp.bfloat16),
                     jnp.zeros((2, BLK), jnp.bfloat16)], axis=0)
        gsrc[pl.dslice(o * BLK, BLK), :] = jnp.transpose(stack16)
        combo_row = lab_row + (1.0 - (sc_row >= thr).astype(jnp.float32)) * 128.0
        comboS[pl.dslice(o, 1), :] = combo_row
        stack = jnp.concatenate([sc_row, jnp.zeros((1, BLK), jnp.float32),
                                 combo_row,
                                 jnp.zeros((5, BLK), jnp.float32)], axis=0)
        origC[pl.dslice(o * BLK, BLK), :] = jnp.transpose(stack)
        return 0

    jax.lax.fori_loop(0, NB, _build_cols, 0, unroll=False)

    tri = (jax.lax.broadcasted_iota(jnp.int32, (BLK, BLK), 1)
           < jax.lax.broadcasted_iota(jnp.int32, (BLK, BLK), 0))

    rrowS[:, :] = jnp.zeros((NB, BLK), jnp.float32)

    def _rank_blk(a, _):
        s_i = origC[pl.dslice(a * BLK, BLK), 0:1]
        c_i = origC[pl.dslice(a * BLK, BLK), 2:3]

        def _pair(bb, acc):
            s_j = scr_ref[0, pl.dslice(bb, 1), :]
            c_j = comboS[pl.dslice(bb, 1), :]
            T = ((c_j < c_i) | ((c_j == c_i) & (s_j > s_i))).astype(jnp.float32)
            colsum = jnp.sum(T, axis=0, keepdims=True)
            rrowS[pl.dslice(bb, 1), :] = rrowS[pl.dslice(bb, 1), :] + (128.0 - colsum)
            return acc + T

        acc = jax.lax.fori_loop(a + 1, NB, _pair,
                                jnp.zeros((BLK, BLK), jnp.float32))
        s_a = scr_ref[0, pl.dslice(a, 1), :]
        c_a = comboS[pl.dslice(a, 1), :]
        full_tie = (c_a == c_i) & (s_a == s_i)
        acc = acc + ((c_a < c_i) | ((c_a == c_i) & (s_a > s_i))
                     | (full_tie & tri)).astype(jnp.float32)
        total = jnp.sum(acc, axis=1, keepdims=True) + \
            jnp.transpose(rrowS[pl.dslice(a, 1), :])
        origC[pl.dslice(a * BLK, BLK), 1:2] = total
        return 0

    jax.lax.fori_loop(0, NB, _rank_blk, 0, unroll=False)

    lane_f = jax.lax.broadcasted_iota(jnp.int32, (1, BLK), 1).astype(jnp.float32)

    def _gather_blk(rr, _):
        tgt = (rr * BLK).astype(jnp.float32) + lane_f

        def _oh(o, _2):
            rank_col = origC[pl.dslice(o * BLK, BLK), 1:2]
            ohT[pl.dslice(o * BLK, BLK), :] = (rank_col == tgt).astype(jnp.bfloat16)
            return 0

        jax.lax.fori_loop(0, NB, _oh, 0, unroll=8)
        rows16 = jax.lax.dot_general(gsrc[:, :], ohT[:, :],
                                     (((0,), (0,)), ((), ())),
                                     preferred_element_type=jnp.float32)
        lo_row = rows16[0:1, :] + rows16[1:2, :] + rows16[2:3, :]
        to_row = rows16[3:4, :] + rows16[4:5, :] + rows16[5:6, :]
        ro_row = rows16[6:7, :] + rows16[7:8, :] + rows16[8:9, :]
        bo_row = rows16[9:10, :] + rows16[10:11, :] + rows16[11:12, :]
        v_row = rows16[12:13, :]
        lab_row = rows16[13:14, :]
        area_row = jnp.maximum(ro_row - lo_row, 0.0) * jnp.maximum(bo_row - to_row, 0.0)
        rows8 = jnp.concatenate([lo_row, to_row, ro_row, bo_row, v_row,
                                 jnp.zeros((1, BLK), jnp.float32), area_row,
                                 jnp.zeros((1, BLK), jnp.float32)], axis=0)
        sortRows[pl.dslice(rr * 8, 8), :] = rows8
        sortC[pl.dslice(rr * BLK, BLK), :] = jnp.transpose(rows8)
        labMM[pl.dslice(rr, 1), 0:1] = jnp.max(
            jnp.where(v_row > 0.5, lab_row, -1.0), axis=1, keepdims=True)
        labMM[pl.dslice(rr, 1), 1:2] = jnp.min(
            jnp.where(v_row > 0.5, lab_row, 1e9), axis=1, keepdims=True)
        return 0

    jax.lax.fori_loop(0, nb_used, _gather_blk, 0, unroll=False)

    lane = jax.lax.broadcasted_iota(jnp.int32, (1, BLK), 1)

    def _nms_blk(q, _):
        cols = sortC[pl.dslice(q * BLK, BLK), :]
        l_i = cols[:, 0:1]
        t_i = cols[:, 1:2]
        r_i = cols[:, 2:3]
        b_i = cols[:, 3:4]
        v_i = cols[:, 4:5]
        area_i = cols[:, 6:7]

        def _iou_rows(rows8):
            l_j = rows8[0:1, :]
            t_j = rows8[1:2, :]
            r_j = rows8[2:3, :]
            b_j = rows8[3:4, :]
            area_j = rows8[6:7, :]
            iw = jnp.maximum(jnp.minimum(r_i, r_j) - jnp.maximum(l_i, l_j), 0.0)
            ih = jnp.maximum(jnp.minimum(b_i, b_j) - jnp.maximum(t_i, t_j), 0.0)
            inter = iw * ih
            union = area_i + area_j - inter
            return inter / jnp.maximum(union, 1e-9)

        def _prev(p, acc):
            rows8 = sortRows[pl.dslice(p * 8, 8), :]
            iou = _iou_rows(rows8)
            keep_j = rows8[5:6, :]
            return jnp.maximum(acc, jnp.where((keep_j > 0.5) & (iou > nmsT), 1.0, 0.0))

        mm = labMM[:, 0:1]
        minq = labMM[pl.dslice(q, 1), 1:2]
        pio = jax.lax.broadcasted_iota(jnp.int32, (NB, 1), 0)
        p0 = jnp.sum(jnp.where((mm < minq) & (pio < q), 1, 0)).astype(jnp.int32)
        p_start = jnp.where(nmsT > 0.0, p0, 0)

        sup_acc = jax.lax.fori_loop(p_start, q, _prev,
                                    jnp.zeros((BLK, BLK), jnp.float32))
        sup = jnp.max(sup_acc, axis=1, keepdims=True)

        iou_in = _iou_rows(sortRows[pl.dslice(q * 8, 8), :])
        v_j = jnp.transpose(v_i)
        conflict = jnp.max(jnp.where(tri & (iou_in > nmsT) & (v_i > 0.5) & (v_j > 0.5),
                                     1.0, 0.0)) > 0.5
        fast = v_i * (1.0 - sup)
        sortRows[pl.dslice(q * 8 + 5, 1), :] = jnp.transpose(fast)

        @pl.when(conflict)
        def _slow():
            iouS[:, :] = iou_in
            auxS[:, 0:1] = sup
            auxS[:, 1:2] = v_i

            def _seq(k, kb):
                row = iouS[pl.dslice(k, 1), :]
                sup_k = auxS[pl.dslice(k, 1), 0:1]
                val_k = auxS[pl.dslice(k, 1), 1:2]
                inblk = jnp.max(jnp.where((lane < k) & (kb > 0.5) & (row > nmsT), 1.0, 0.0),
                                axis=1, keepdims=True)
                kept = val_k * (1.0 - jnp.maximum(sup_k, inblk))
                return jnp.where(lane == k, kept, kb)

            kb = jax.lax.fori_loop(0, BLK, _seq, jnp.zeros((1, BLK), jnp.float32))
            sortRows[pl.dslice(q * 8 + 5, 1), :] = kb
        return 0

    jax.lax.fori_loop(0, nb_used, _nms_blk, 0, unroll=False)

    def _unperm(o, _):
        rank_col = origC[pl.dslice(o * BLK, BLK), 1:2]

        def _scan(s, acc):
            keep_row = sortRows[pl.dslice(s * 8 + 5, 1), :]
            srcpos = (s * BLK).astype(jnp.float32) + lane_f
            return jnp.maximum(acc, jnp.where((rank_col == srcpos) & (keep_row > 0.5),
                                              1.0, 0.0))

        acc = jax.lax.fori_loop(0, nb_used, _scan, jnp.zeros((BLK, BLK), jnp.float32))
        krow = jnp.transpose(jnp.max(acc, axis=1, keepdims=True))
        kl_ref[0, pl.dslice(o, 1), :] = lr_ref[0, pl.dslice(o, 1), :] * scale * krow
        kt_ref[0, pl.dslice(o, 1), :] = tr_ref[0, pl.dslice(o, 1), :] * scale * krow
        kr_ref[0, pl.dslice(o, 1), :] = rr_ref[0, pl.dslice(o, 1), :] * scale * krow
        kb_ref[0, pl.dslice(o, 1), :] = br_ref[0, pl.dslice(o, 1), :] * scale * krow
        ksc_ref[0, pl.dslice(o, 1), :] = scr_ref[0, pl.dslice(o, 1), :] * krow
        keep_ref[0, pl.dslice(o, 1), :] = krow
        return 0

    jax.lax.fori_loop(0, NB, _unperm, 0, unroll=False)


def kernel(features, anchors, image_sizes, image_sizes_ori, score_thresh, nms_thresh):
    B = features.shape[0]
    ft = jnp.transpose(features, (0, 2, 1))
    ft = jnp.pad(ft, ((0, 0), (0, 3), (0, NPAD - N)))
    at = jnp.pad(jnp.transpose(anchors, (1, 0)), ((0, 3), (0, NPAD - N)))
    st = jnp.reshape(jnp.asarray(score_thresh, jnp.float32), (1, 1))
    nt = jnp.reshape(jnp.asarray(nms_thresh, jnp.float32), (1, 1))

    f32 = jnp.float32
    sc, lab, l, t, r, b = pl.pallas_call(
        _prep_body,
        grid=(B,),
        in_specs=[
            pl.BlockSpec((1, 88, NPAD), lambda i: (i, 0, 0)),
            pl.BlockSpec((8, NPAD), lambda i: (0, 0)),
            pl.BlockSpec((1, 1, 2), lambda i: (i, 0, 0), memory_space=pltpu.SMEM),
        ],
        out_specs=[pl.BlockSpec((1, 1, NPAD), lambda i: (i, 0, 0))] * 6,
        out_shape=[
            jax.ShapeDtypeStruct((B, 1, NPAD), f32),
            jax.ShapeDtypeStruct((B, 1, NPAD), jnp.int32),
            jax.ShapeDtypeStruct((B, 1, NPAD), f32),
            jax.ShapeDtypeStruct((B, 1, NPAD), f32),
            jax.ShapeDtypeStruct((B, 1, NPAD), f32),
            jax.ShapeDtypeStruct((B, 1, NPAD), f32),
        ],
    )(ft, at, image_sizes.reshape(B, 1, 2))

    scr = sc.reshape(B, NB, BLK)
    lr = l.reshape(B, NB, BLK)
    tr = t.reshape(B, NB, BLK)
    rr = r.reshape(B, NB, BLK)
    br = b.reshape(B, NB, BLK)
    labf = lab.astype(f32).reshape(B, NB, BLK)

    row_spec = pl.BlockSpec((1, NB, BLK), lambda i: (i, 0, 0))
    smem2 = pl.BlockSpec((1, 1, 2), lambda i: (i, 0, 0), memory_space=pltpu.SMEM)
    smem1 = pl.BlockSpec((1, 1), lambda i: (0, 0), memory_space=pltpu.SMEM)
    outs = pl.pallas_call(
        _nms_body,
        grid=(B,),
        in_specs=[row_spec] * 6 + [smem2, smem2, smem1, smem1],
        out_specs=[row_spec] * 6,
        out_shape=[jax.ShapeDtypeStruct((B, NB, BLK), f32)] * 6,
        scratch_shapes=[
            pltpu.VMEM((NPAD, 8), f32),
            pltpu.VMEM((NPAD, 16), jnp.bfloat16),
            pltpu.VMEM((NPAD, BLK), jnp.bfloat16),
            pltpu.VMEM((NPAD, 8), f32),
            pltpu.VMEM((NB * 8, BLK), f32),
            pltpu.VMEM((NB, BLK), f32),
            pltpu.VMEM((NB, BLK), f32),
            pltpu.VMEM((NB, 8), f32),
            pltpu.VMEM((BLK, BLK), f32),
            pltpu.VMEM((BLK, 8), f32),
        ],
    )(scr, lr, tr, rr, br, labf, image_sizes.reshape(B, 1, 2),
      image_sizes_ori.reshape(B, 1, 2), st, nt)
    kl, kt, kr, kb, ksc, keepf = outs

    out_boxes = jnp.stack([
        kl.reshape(B, NPAD)[:, :N], kt.reshape(B, NPAD)[:, :N],
        kr.reshape(B, NPAD)[:, :N], kb.reshape(B, NPAD)[:, :N]], axis=-1)
    out_scores = ksc.reshape(B, NPAD)[:, :N]
    out_labels = lab.reshape(B, NPAD)[:, :N]
    out_keep = keepf.reshape(B, NPAD)[:, :N] > 0.5
    return (out_boxes, out_scores, out_labels, out_keep)


# speedup vs baseline: 1.4200x; 1.2907x over previous
import jax
import jax.numpy as jnp
from jax.experimental import pallas as pl
from jax.experimental.pallas import tpu as pltpu

N = 5000
NPAD = 5120
BLK = 128
NB = NPAD // BLK
NCLS = 80


def _prep_body(ft_ref, at_ref, isz_ref, sc_ref, lab_ref, l_ref, t_ref, r_ref, b_ref):
    f = ft_ref[0]
    H = isz_ref[0, 0, 0].astype(jnp.float32)
    W = isz_ref[0, 0, 1].astype(jnp.float32)
    cls = f[5:5 + NCLS, :]
    mx = jnp.max(cls, axis=0, keepdims=True)
    rowio = jax.lax.broadcasted_iota(jnp.int32, (NCLS, NPAD), 0)
    lab = jnp.min(jnp.where(cls == mx, rowio, jnp.int32(2 ** 30)), axis=0, keepdims=True)
    sc = mx * f[4:5, :]

    cell_x = at_ref[0:1, :]
    cell_y = at_ref[1:2, :]
    stride = at_ref[2:3, :]
    w_a = at_ref[3:4, :]
    h_a = at_ref[4:5, :]
    cx = (f[0:1, :] + cell_x) * stride
    cy = (f[1:2, :] + cell_y) * stride
    w = w_a * jnp.exp(f[2:3, :])
    h = h_a * jnp.exp(f[3:4, :])
    l_un = cx - w / 2.0
    t_un = cy - h / 2.0
    r_un = l_un + w
    b_un = t_un + h
    l = jnp.clip(l_un, 0.0, W - 1.0)
    r = jnp.clip(r_un, 0.0, W - 1.0)
    t = jnp.clip(t_un, 0.0, H - 1.0)
    b = jnp.clip(b_un, 0.0, H - 1.0)

    sc_ref[0] = sc
    lab_ref[0] = lab
    l_ref[0] = l
    t_ref[0] = t
    r_ref[0] = r
    b_ref[0] = b


def _nms_body(scr_ref, lr_ref, tr_ref, rr_ref, br_ref, labf_ref,
              isz_ref, iszo_ref, st_ref, nt_ref,
              kl_ref, kt_ref, kr_ref, kb_ref, ksc_ref, keep_ref,
              origC, gsrc, ohT, sortC, sortRows, rrowS, comboS, labMM, iouS, auxS):
    thr = st_ref[0, 0]
    nmsT = nt_ref[0, 0]
    H = isz_ref[0, 0, 0].astype(jnp.float32)
    scale = iszo_ref[0, 0, 0].astype(jnp.float32) / H

    sc = scr_ref[0]
    maxc = jnp.max(jnp.maximum(rr_ref[0], br_ref[0])) + 1.0
    nvalid = jnp.sum((sc >= thr).astype(jnp.int32))
    nb_used = (nvalid + (BLK - 1)) // BLK

    def _split3(x):
        hi = x.astype(jnp.bfloat16)
        r1 = x - hi.astype(jnp.float32)
        mid = r1.astype(jnp.bfloat16)
        r2 = r1 - mid.astype(jnp.float32)
        return [hi, mid, r2.astype(jnp.bfloat16)]

    def _build_cols(o, _):
        lab_row = labf_ref[0, pl.dslice(o, 1), :]
        off_row = lab_row * maxc
        sc_row = scr_ref[0, pl.dslice(o, 1), :]
        parts = (_split3(lr_ref[0, pl.dslice(o, 1), :] + off_row)
                 + _split3(tr_ref[0, pl.dslice(o, 1), :] + off_row)
                 + _split3(rr_ref[0, pl.dslice(o, 1), :] + off_row)
                 + _split3(br_ref[0, pl.dslice(o, 1), :] + off_row))
        stack16 = jnp.concatenate(
            parts + [(sc_row >= thr).astype(jnp.bfloat16),
                     lab_row.astype(jnp.bfloat16),
                     jnp.zeros((2, BLK), jnp.bfloat16)], axis=0)
        gsrc[pl.dslice(o * BLK, BLK), :] = jnp.transpose(stack16)
        combo_row = lab_row + (1.0 - (sc_row >= thr).astype(jnp.float32)) * 128.0
        comboS[pl.dslice(o, 1), :] = combo_row
        stack = jnp.concatenate([sc_row, jnp.zeros((1, BLK), jnp.float32),
                                 combo_row,
                                 jnp.zeros((5, BLK), jnp.float32)], axis=0)
        origC[pl.dslice(o * BLK, BLK), :] = jnp.transpose(stack)
        return 0

    jax.lax.fori_loop(0, NB, _build_cols, 0, unroll=False)

    tri = (jax.lax.broadcasted_iota(jnp.int32, (BLK, BLK), 1)
           < jax.lax.broadcasted_iota(jnp.int32, (BLK, BLK), 0))

    rrowS[:, :] = jnp.zeros((NB, BLK), jnp.float32)

    def _rank_blk(a, _):
        s_i = origC[pl.dslice(a * BLK, BLK), 0:1]
        c_i = origC[pl.dslice(a * BLK, BLK), 2:3]

        def _pair(bb, acc):
            s_j = scr_ref[0, pl.dslice(bb, 1), :]
            c_j = comboS[pl.dslice(bb, 1), :]
            T = ((c_j < c_i) | ((c_j == c_i) & (s_j > s_i))).astype(jnp.float32)
            colsum = jnp.sum(T, axis=0, keepdims=True)
            rrowS[pl.dslice(bb, 1), :] = rrowS[pl.dslice(bb, 1), :] + (128.0 - colsum)
            return acc + T

        acc = jax.lax.fori_loop(a + 1, NB, _pair,
                                jnp.zeros((BLK, BLK), jnp.float32))
        s_a = scr_ref[0, pl.dslice(a, 1), :]
        c_a = comboS[pl.dslice(a, 1), :]
        full_tie = (c_a == c_i) & (s_a == s_i)
        acc = acc + ((c_a < c_i) | ((c_a == c_i) & (s_a > s_i))
                     | (full_tie & tri)).astype(jnp.float32)
        total = jnp.sum(acc, axis=1, keepdims=True) + \
            jnp.transpose(rrowS[pl.dslice(a, 1), :])
        origC[pl.dslice(a * BLK, BLK), 1:2] = total
        return 0

    jax.lax.fori_loop(0, NB, _rank_blk, 0, unroll=False)

    GW = 4
    lane_f = jax.lax.broadcasted_iota(jnp.int32, (1, BLK), 1).astype(jnp.float32)
    lane_w = jax.lax.broadcasted_iota(jnp.int32, (1, GW * BLK), 1).astype(jnp.float32)

    def _gather_blk(r4, _):
        tgt = (r4 * (GW * BLK)).astype(jnp.float32) + lane_w

        def _oh(o, _2):
            rank_col = origC[pl.dslice(o * BLK, BLK), 1:2]
            ohT[pl.dslice(o * BLK, BLK), :] = (rank_col == tgt).astype(jnp.bfloat16)
            return 0

        jax.lax.fori_loop(0, NB, _oh, 0, unroll=8)
        rows16w = jax.lax.dot_general(gsrc[:, :], ohT[:, :],
                                      (((0,), (0,)), ((), ())),
                                      preferred_element_type=jnp.float32)
        for c4 in range(GW):
            rows16 = rows16w[:, c4 * BLK:(c4 + 1) * BLK]
            rr = r4 * GW + c4
            lo_row = rows16[0:1, :] + rows16[1:2, :] + rows16[2:3, :]
            to_row = rows16[3:4, :] + rows16[4:5, :] + rows16[5:6, :]
            ro_row = rows16[6:7, :] + rows16[7:8, :] + rows16[8:9, :]
            bo_row = rows16[9:10, :] + rows16[10:11, :] + rows16[11:12, :]
            v_row = rows16[12:13, :]
            lab_row = rows16[13:14, :]
            area_row = jnp.maximum(ro_row - lo_row, 0.0) * jnp.maximum(bo_row - to_row, 0.0)
            rows8 = jnp.concatenate([lo_row, to_row, ro_row, bo_row, v_row,
                                     jnp.zeros((1, BLK), jnp.float32), area_row,
                                     jnp.zeros((1, BLK), jnp.float32)], axis=0)
            sortRows[pl.dslice(rr * 8, 8), :] = rows8
            sortC[pl.dslice(rr * BLK, BLK), :] = jnp.transpose(rows8)
            labMM[pl.dslice(rr, 1), 0:1] = jnp.max(
                jnp.where(v_row > 0.5, lab_row, -1.0), axis=1, keepdims=True)
            labMM[pl.dslice(rr, 1), 1:2] = jnp.min(
                jnp.where(v_row > 0.5, lab_row, 1e9), axis=1, keepdims=True)
        return 0

    jax.lax.fori_loop(0, (nb_used + GW - 1) // GW, _gather_blk, 0, unroll=False)

    lane = jax.lax.broadcasted_iota(jnp.int32, (1, BLK), 1)

    def _nms_blk(q, _):
        cols = sortC[pl.dslice(q * BLK, BLK), :]
        l_i = cols[:, 0:1]
        t_i = cols[:, 1:2]
        r_i = cols[:, 2:3]
        b_i = cols[:, 3:4]
        v_i = cols[:, 4:5]
        area_i = cols[:, 6:7]

        def _iou_rows(rows8):
            l_j = rows8[0:1, :]
            t_j = rows8[1:2, :]
            r_j = rows8[2:3, :]
            b_j = rows8[3:4, :]
            area_j = rows8[6:7, :]
            iw = jnp.maximum(jnp.minimum(r_i, r_j) - jnp.maximum(l_i, l_j), 0.0)
            ih = jnp.maximum(jnp.minimum(b_i, b_j) - jnp.maximum(t_i, t_j), 0.0)
            inter = iw * ih
            union = area_i + area_j - inter
            return inter / jnp.maximum(union, 1e-9)

        def _prev(p, acc):
            rows8 = sortRows[pl.dslice(p * 8, 8), :]
            iou = _iou_rows(rows8)
            keep_j = rows8[5:6, :]
            return jnp.maximum(acc, jnp.where((keep_j > 0.5) & (iou > nmsT), 1.0, 0.0))

        mm = labMM[:, 0:1]
        minq = labMM[pl.dslice(q, 1), 1:2]
        pio = jax.lax.broadcasted_iota(jnp.int32, (NB, 1), 0)
        p0 = jnp.sum(jnp.where((mm < minq) & (pio < q), 1, 0)).astype(jnp.int32)
        p_start = jnp.where(nmsT > 0.0, p0, 0)

        sup_acc = jax.lax.fori_loop(p_start, q, _prev,
                                    jnp.zeros((BLK, BLK), jnp.float32))
        sup = jnp.max(sup_acc, axis=1, keepdims=True)

        iou_in = _iou_rows(sortRows[pl.dslice(q * 8, 8), :])
        v_j = jnp.transpose(v_i)
        conflict = jnp.max(jnp.where(tri & (iou_in > nmsT) & (v_i > 0.5) & (v_j > 0.5),
                                     1.0, 0.0)) > 0.5
        fast = v_i * (1.0 - sup)
        sortRows[pl.dslice(q * 8 + 5, 1), :] = jnp.transpose(fast)

        @pl.when(conflict)
        def _slow():
            iouS[:, :] = iou_in
            auxS[:, 0:1] = sup
            auxS[:, 1:2] = v_i

            def _seq(k, kb):
                row = iouS[pl.dslice(k, 1), :]
                sup_k = auxS[pl.dslice(k, 1), 0:1]
                val_k = auxS[pl.dslice(k, 1), 1:2]
                inblk = jnp.max(jnp.where((lane < k) & (kb > 0.5) & (row > nmsT), 1.0, 0.0),
                                axis=1, keepdims=True)
                kept = val_k * (1.0 - jnp.maximum(sup_k, inblk))
                return jnp.where(lane == k, kept, kb)

            kb = jax.lax.fori_loop(0, BLK, _seq, jnp.zeros((1, BLK), jnp.float32))
            sortRows[pl.dslice(q * 8 + 5, 1), :] = kb
        return 0

    jax.lax.fori_loop(0, nb_used, _nms_blk, 0, unroll=False)

    def _unperm(o, _):
        rank_col = origC[pl.dslice(o * BLK, BLK), 1:2]

        def _scan(s4, acc):
            keep_row = jnp.concatenate(
                [sortRows[pl.dslice((s4 * GW + k) * 8 + 5, 1), :] for k in range(GW)],
                axis=1)
            srcpos = (s4 * (GW * BLK)).astype(jnp.float32) + lane_w
            hit = jnp.where((rank_col == srcpos) & (keep_row > 0.5), 1.0, 0.0)
            return jnp.maximum(acc, jnp.max(hit, axis=1, keepdims=True))

        acc = jax.lax.fori_loop(0, (nb_used + GW - 1) // GW, _scan,
                                jnp.zeros((BLK, 1), jnp.float32))
        krow = jnp.transpose(acc)
        kl_ref[0, pl.dslice(o, 1), :] = lr_ref[0, pl.dslice(o, 1), :] * scale * krow
        kt_ref[0, pl.dslice(o, 1), :] = tr_ref[0, pl.dslice(o, 1), :] * scale * krow
        kr_ref[0, pl.dslice(o, 1), :] = rr_ref[0, pl.dslice(o, 1), :] * scale * krow
        kb_ref[0, pl.dslice(o, 1), :] = br_ref[0, pl.dslice(o, 1), :] * scale * krow
        ksc_ref[0, pl.dslice(o, 1), :] = scr_ref[0, pl.dslice(o, 1), :] * krow
        keep_ref[0, pl.dslice(o, 1), :] = krow
        return 0

    jax.lax.fori_loop(0, NB, _unperm, 0, unroll=False)


def kernel(features, anchors, image_sizes, image_sizes_ori, score_thresh, nms_thresh):
    B = features.shape[0]
    ft = jnp.transpose(features, (0, 2, 1))
    ft = jnp.pad(ft, ((0, 0), (0, 3), (0, NPAD - N)))
    at = jnp.pad(jnp.transpose(anchors, (1, 0)), ((0, 3), (0, NPAD - N)))
    st = jnp.reshape(jnp.asarray(score_thresh, jnp.float32), (1, 1))
    nt = jnp.reshape(jnp.asarray(nms_thresh, jnp.float32), (1, 1))

    f32 = jnp.float32
    sc, lab, l, t, r, b = pl.pallas_call(
        _prep_body,
        grid=(B,),
        in_specs=[
            pl.BlockSpec((1, 88, NPAD), lambda i: (i, 0, 0)),
            pl.BlockSpec((8, NPAD), lambda i: (0, 0)),
            pl.BlockSpec((1, 1, 2), lambda i: (i, 0, 0), memory_space=pltpu.SMEM),
        ],
        out_specs=[pl.BlockSpec((1, 1, NPAD), lambda i: (i, 0, 0))] * 6,
        out_shape=[
            jax.ShapeDtypeStruct((B, 1, NPAD), f32),
            jax.ShapeDtypeStruct((B, 1, NPAD), jnp.int32),
            jax.ShapeDtypeStruct((B, 1, NPAD), f32),
            jax.ShapeDtypeStruct((B, 1, NPAD), f32),
            jax.ShapeDtypeStruct((B, 1, NPAD), f32),
            jax.ShapeDtypeStruct((B, 1, NPAD), f32),
        ],
    )(ft, at, image_sizes.reshape(B, 1, 2))

    scr = sc.reshape(B, NB, BLK)
    lr = l.reshape(B, NB, BLK)
    tr = t.reshape(B, NB, BLK)
    rr = r.reshape(B, NB, BLK)
    br = b.reshape(B, NB, BLK)
    labf = lab.astype(f32).reshape(B, NB, BLK)

    row_spec = pl.BlockSpec((1, NB, BLK), lambda i: (i, 0, 0))
    smem2 = pl.BlockSpec((1, 1, 2), lambda i: (i, 0, 0), memory_space=pltpu.SMEM)
    smem1 = pl.BlockSpec((1, 1), lambda i: (0, 0), memory_space=pltpu.SMEM)
    outs = pl.pallas_call(
        _nms_body,
        grid=(B,),
        in_specs=[row_spec] * 6 + [smem2, smem2, smem1, smem1],
        out_specs=[row_spec] * 6,
        out_shape=[jax.ShapeDtypeStruct((B, NB, BLK), f32)] * 6,
        scratch_shapes=[
            pltpu.VMEM((NPAD, 8), f32),
            pltpu.VMEM((NPAD, 16), jnp.bfloat16),
            pltpu.VMEM((NPAD, 4 * BLK), jnp.bfloat16),
            pltpu.VMEM((NPAD, 8), f32),
            pltpu.VMEM((NB * 8, BLK), f32),
            pltpu.VMEM((NB, BLK), f32),
            pltpu.VMEM((NB, BLK), f32),
            pltpu.VMEM((NB, 8), f32),
            pltpu.VMEM((BLK, BLK), f32),
            pltpu.VMEM((BLK, 8), f32),
        ],
    )(scr, lr, tr, rr, br, labf, image_sizes.reshape(B, 1, 2),
      image_sizes_ori.reshape(B, 1, 2), st, nt)
    kl, kt, kr, kb, ksc, keepf = outs

    out_boxes = jnp.stack([
        kl.reshape(B, NPAD)[:, :N], kt.reshape(B, NPAD)[:, :N],
        kr.reshape(B, NPAD)[:, :N], kb.reshape(B, NPAD)[:, :N]], axis=-1)
    out_scores = ksc.reshape(B, NPAD)[:, :N]
    out_labels = lab.reshape(B, NPAD)[:, :N]
    out_keep = keepf.reshape(B, NPAD)[:, :N] > 0.5
    return (out_boxes, out_scores, out_labels, out_keep)


# speedup vs baseline: 2.0349x; 1.4330x over previous
import jax
import jax.numpy as jnp
from jax.experimental import pallas as pl
from jax.experimental.pallas import tpu as pltpu

N = 5000
NPAD = 5120
BLK = 128
NB = NPAD // BLK
NCLS = 80


def _prep_body(ft_ref, at_ref, isz_ref, sc_ref, lab_ref, l_ref, t_ref, r_ref, b_ref):
    f = ft_ref[0]
    H = isz_ref[0, 0, 0].astype(jnp.float32)
    W = isz_ref[0, 0, 1].astype(jnp.float32)
    cls = f[5:5 + NCLS, :]
    mx = jnp.max(cls, axis=0, keepdims=True)
    rowio = jax.lax.broadcasted_iota(jnp.int32, (NCLS, NPAD), 0)
    lab = jnp.min(jnp.where(cls == mx, rowio, jnp.int32(2 ** 30)), axis=0, keepdims=True)
    sc = mx * f[4:5, :]

    cell_x = at_ref[0:1, :]
    cell_y = at_ref[1:2, :]
    stride = at_ref[2:3, :]
    w_a = at_ref[3:4, :]
    h_a = at_ref[4:5, :]
    cx = (f[0:1, :] + cell_x) * stride
    cy = (f[1:2, :] + cell_y) * stride
    w = w_a * jnp.exp(f[2:3, :])
    h = h_a * jnp.exp(f[3:4, :])
    l_un = cx - w / 2.0
    t_un = cy - h / 2.0
    r_un = l_un + w
    b_un = t_un + h
    l = jnp.clip(l_un, 0.0, W - 1.0)
    r = jnp.clip(r_un, 0.0, W - 1.0)
    t = jnp.clip(t_un, 0.0, H - 1.0)
    b = jnp.clip(b_un, 0.0, H - 1.0)

    sc_ref[0] = sc
    lab_ref[0] = lab
    l_ref[0] = l
    t_ref[0] = t
    r_ref[0] = r
    b_ref[0] = b


def _nms_body(scr_ref, lr_ref, tr_ref, rr_ref, br_ref, labf_ref,
              isz_ref, iszo_ref, st_ref, nt_ref,
              kl_ref, kt_ref, kr_ref, kb_ref, ksc_ref, keep_ref,
              origC, gsrc, ohT, sortC, sortRows, rrowS, comboS, labMM, iouS, auxS):
    thr = st_ref[0, 0]
    nmsT = nt_ref[0, 0]
    H = isz_ref[0, 0, 0].astype(jnp.float32)
    scale = iszo_ref[0, 0, 0].astype(jnp.float32) / H

    sc = scr_ref[0]
    maxc = jnp.max(jnp.maximum(rr_ref[0], br_ref[0])) + 1.0
    nvalid = jnp.sum((sc >= thr).astype(jnp.int32))
    nb_used = (nvalid + (BLK - 1)) // BLK

    def _split3(x):
        hi = x.astype(jnp.bfloat16)
        r1 = x - hi.astype(jnp.float32)
        mid = r1.astype(jnp.bfloat16)
        r2 = r1 - mid.astype(jnp.float32)
        return [hi, mid, r2.astype(jnp.bfloat16)]

    def _build_cols(o, _):
        lab_row = labf_ref[0, pl.dslice(o, 1), :]
        off_row = lab_row * maxc
        sc_row = scr_ref[0, pl.dslice(o, 1), :]
        parts = (_split3(lr_ref[0, pl.dslice(o, 1), :] + off_row)
                 + _split3(tr_ref[0, pl.dslice(o, 1), :] + off_row)
                 + _split3(rr_ref[0, pl.dslice(o, 1), :] + off_row)
                 + _split3(br_ref[0, pl.dslice(o, 1), :] + off_row))
        stack16 = jnp.concatenate(
            parts + [(sc_row >= thr).astype(jnp.bfloat16),
                     lab_row.astype(jnp.bfloat16),
                     jnp.zeros((2, BLK), jnp.bfloat16)], axis=0)
        gsrc[pl.dslice(o * BLK, BLK), :] = jnp.transpose(stack16)
        combo_row = lab_row + (1.0 - (sc_row >= thr).astype(jnp.float32)) * 128.0
        comboS[pl.dslice(o, 1), :] = combo_row
        stack = jnp.concatenate([sc_row, jnp.zeros((1, BLK), jnp.float32),
                                 combo_row,
                                 jnp.zeros((5, BLK), jnp.float32)], axis=0)
        origC[pl.dslice(o * BLK, BLK), :] = jnp.transpose(stack)
        return 0

    jax.lax.fori_loop(0, NB, _build_cols, 0, unroll=False)

    tri = (jax.lax.broadcasted_iota(jnp.int32, (BLK, BLK), 1)
           < jax.lax.broadcasted_iota(jnp.int32, (BLK, BLK), 0))

    rrowS[:, :] = jnp.zeros((NB, BLK), jnp.float32)

    def _rank_blk(a, _):
        s_i = jnp.broadcast_to(origC[pl.dslice(a * BLK, BLK), 0:1], (BLK, BLK))
        c_i = jnp.broadcast_to(origC[pl.dslice(a * BLK, BLK), 2:3], (BLK, BLK))

        def _pair(bb, acc):
            s_j = scr_ref[0, pl.dslice(bb, 1), :]
            c_j = comboS[pl.dslice(bb, 1), :]
            T = ((c_j < c_i) | ((c_j == c_i) & (s_j > s_i))).astype(jnp.float32)
            colsum = jnp.sum(T, axis=0, keepdims=True)
            rrowS[pl.dslice(bb, 1), :] = rrowS[pl.dslice(bb, 1), :] + (128.0 - colsum)
            return acc + T

        acc = jax.lax.fori_loop(a + 1, NB, _pair,
                                jnp.zeros((BLK, BLK), jnp.float32))
        s_a = scr_ref[0, pl.dslice(a, 1), :]
        c_a = comboS[pl.dslice(a, 1), :]
        full_tie = (c_a == c_i) & (s_a == s_i)
        acc = acc + ((c_a < c_i) | ((c_a == c_i) & (s_a > s_i))
                     | (full_tie & tri)).astype(jnp.float32)
        total = jnp.sum(acc, axis=1, keepdims=True) + \
            jnp.transpose(rrowS[pl.dslice(a, 1), :])
        origC[pl.dslice(a * BLK, BLK), 1:2] = total
        return 0

    jax.lax.fori_loop(0, NB, _rank_blk, 0, unroll=False)

    GW = 4
    lane_f = jax.lax.broadcasted_iota(jnp.int32, (1, BLK), 1).astype(jnp.float32)
    lane_w = jax.lax.broadcasted_iota(jnp.int32, (1, GW * BLK), 1).astype(jnp.float32)

    def _gather_blk(r4, _):
        tgt = (r4 * (GW * BLK)).astype(jnp.float32) + lane_w

        def _oh(o, _2):
            rank_col = origC[pl.dslice(o * BLK, BLK), 1:2]
            ohT[pl.dslice(o * BLK, BLK), :] = (rank_col == tgt).astype(jnp.bfloat16)
            return 0

        jax.lax.fori_loop(0, NB, _oh, 0, unroll=8)
        rows16w = jax.lax.dot_general(gsrc[:, :], ohT[:, :],
                                      (((0,), (0,)), ((), ())),
                                      preferred_element_type=jnp.float32)
        for c4 in range(GW):
            rows16 = rows16w[:, c4 * BLK:(c4 + 1) * BLK]
            rr = r4 * GW + c4
            lo_row = rows16[0:1, :] + rows16[1:2, :] + rows16[2:3, :]
            to_row = rows16[3:4, :] + rows16[4:5, :] + rows16[5:6, :]
            ro_row = rows16[6:7, :] + rows16[7:8, :] + rows16[8:9, :]
            bo_row = rows16[9:10, :] + rows16[10:11, :] + rows16[11:12, :]
            v_row = rows16[12:13, :]
            lab_row = rows16[13:14, :]
            area_row = jnp.maximum(ro_row - lo_row, 0.0) * jnp.maximum(bo_row - to_row, 0.0)
            rows8 = jnp.concatenate([lo_row, to_row, ro_row, bo_row, v_row,
                                     jnp.zeros((1, BLK), jnp.float32), area_row,
                                     jnp.zeros((1, BLK), jnp.float32)], axis=0)
            sortRows[pl.dslice(rr * 8, 8), :] = rows8
            sortC[pl.dslice(rr * BLK, BLK), :] = jnp.transpose(rows8)
            labMM[pl.dslice(rr, 1), 0:1] = jnp.max(
                jnp.where(v_row > 0.5, lab_row, -1.0), axis=1, keepdims=True)
            labMM[pl.dslice(rr, 1), 1:2] = jnp.min(
                jnp.where(v_row > 0.5, lab_row, 1e9), axis=1, keepdims=True)
        return 0

    jax.lax.fori_loop(0, (nb_used + GW - 1) // GW, _gather_blk, 0, unroll=False)

    lane = jax.lax.broadcasted_iota(jnp.int32, (1, BLK), 1)

    def _nms_blk(q, _):
        cols = sortC[pl.dslice(q * BLK, BLK), :]
        l_i = cols[:, 0:1]
        t_i = cols[:, 1:2]
        r_i = cols[:, 2:3]
        b_i = cols[:, 3:4]
        v_i = cols[:, 4:5]
        area_i = cols[:, 6:7]

        def _iou_rows(rows8):
            l_j = rows8[0:1, :]
            t_j = rows8[1:2, :]
            r_j = rows8[2:3, :]
            b_j = rows8[3:4, :]
            area_j = rows8[6:7, :]
            iw = jnp.maximum(jnp.minimum(r_i, r_j) - jnp.maximum(l_i, l_j), 0.0)
            ih = jnp.maximum(jnp.minimum(b_i, b_j) - jnp.maximum(t_i, t_j), 0.0)
            inter = iw * ih
            union = area_i + area_j - inter
            return inter / jnp.maximum(union, 1e-9)

        def _prev(p, acc):
            rows8 = sortRows[pl.dslice(p * 8, 8), :]
            iou = _iou_rows(rows8)
            keep_j = rows8[5:6, :]
            return jnp.maximum(acc, jnp.where((keep_j > 0.5) & (iou > nmsT), 1.0, 0.0))

        mm = labMM[:, 0:1]
        minq = labMM[pl.dslice(q, 1), 1:2]
        pio = jax.lax.broadcasted_iota(jnp.int32, (NB, 1), 0)
        p0 = jnp.sum(jnp.where((mm < minq) & (pio < q), 1, 0)).astype(jnp.int32)
        p_start = jnp.where(nmsT > 0.0, p0, 0)

        sup_acc = jax.lax.fori_loop(p_start, q, _prev,
                                    jnp.zeros((BLK, BLK), jnp.float32))
        sup = jnp.max(sup_acc, axis=1, keepdims=True)

        iou_in = _iou_rows(sortRows[pl.dslice(q * 8, 8), :])
        v_j = jnp.transpose(v_i)
        conflict = jnp.max(jnp.where(tri & (iou_in > nmsT) & (v_i > 0.5) & (v_j > 0.5),
                                     1.0, 0.0)) > 0.5
        fast = v_i * (1.0 - sup)
        sortRows[pl.dslice(q * 8 + 5, 1), :] = jnp.transpose(fast)

        @pl.when(conflict)
        def _slow():
            iouS[:, :] = iou_in
            auxS[:, 0:1] = sup
            auxS[:, 1:2] = v_i

            def _seq(k, kb):
                row = iouS[pl.dslice(k, 1), :]
                sup_k = auxS[pl.dslice(k, 1), 0:1]
                val_k = auxS[pl.dslice(k, 1), 1:2]
                inblk = jnp.max(jnp.where((lane < k) & (kb > 0.5) & (row > nmsT), 1.0, 0.0),
                                axis=1, keepdims=True)
                kept = val_k * (1.0 - jnp.maximum(sup_k, inblk))
                return jnp.where(lane == k, kept, kb)

            kb = jax.lax.fori_loop(0, BLK, _seq, jnp.zeros((1, BLK), jnp.float32))
            sortRows[pl.dslice(q * 8 + 5, 1), :] = kb
        return 0

    jax.lax.fori_loop(0, nb_used, _nms_blk, 0, unroll=False)

    def _unperm(o, _):
        rank_col = origC[pl.dslice(o * BLK, BLK), 1:2]

        def _scan(s4, acc):
            keep_row = jnp.concatenate(
                [sortRows[pl.dslice((s4 * GW + k) * 8 + 5, 1), :] for k in range(GW)],
                axis=1)
            srcpos = (s4 * (GW * BLK)).astype(jnp.float32) + lane_w
            hit = jnp.where((rank_col == srcpos) & (keep_row > 0.5), 1.0, 0.0)
            return jnp.maximum(acc, jnp.max(hit, axis=1, keepdims=True))

        acc = jax.lax.fori_loop(0, (nb_used + GW - 1) // GW, _scan,
                                jnp.zeros((BLK, 1), jnp.float32))
        krow = jnp.transpose(acc)
        kl_ref[0, pl.dslice(o, 1), :] = lr_ref[0, pl.dslice(o, 1), :] * scale * krow
        kt_ref[0, pl.dslice(o, 1), :] = tr_ref[0, pl.dslice(o, 1), :] * scale * krow
        kr_ref[0, pl.dslice(o, 1), :] = rr_ref[0, pl.dslice(o, 1), :] * scale * krow
        kb_ref[0, pl.dslice(o, 1), :] = br_ref[0, pl.dslice(o, 1), :] * scale * krow
        ksc_ref[0, pl.dslice(o, 1), :] = scr_ref[0, pl.dslice(o, 1), :] * krow
        keep_ref[0, pl.dslice(o, 1), :] = krow
        return 0

    jax.lax.fori_loop(0, NB, _unperm, 0, unroll=False)


def kernel(features, anchors, image_sizes, image_sizes_ori, score_thresh, nms_thresh):
    B = features.shape[0]
    ft = jnp.transpose(features, (0, 2, 1))
    ft = jnp.pad(ft, ((0, 0), (0, 3), (0, NPAD - N)))
    at = jnp.pad(jnp.transpose(anchors, (1, 0)), ((0, 3), (0, NPAD - N)))
    st = jnp.reshape(jnp.asarray(score_thresh, jnp.float32), (1, 1))
    nt = jnp.reshape(jnp.asarray(nms_thresh, jnp.float32), (1, 1))

    f32 = jnp.float32
    sc, lab, l, t, r, b = pl.pallas_call(
        _prep_body,
        grid=(B,),
        in_specs=[
            pl.BlockSpec((1, 88, NPAD), lambda i: (i, 0, 0)),
            pl.BlockSpec((8, NPAD), lambda i: (0, 0)),
            pl.BlockSpec((1, 1, 2), lambda i: (i, 0, 0), memory_space=pltpu.SMEM),
        ],
        out_specs=[pl.BlockSpec((1, 1, NPAD), lambda i: (i, 0, 0))] * 6,
        out_shape=[
            jax.ShapeDtypeStruct((B, 1, NPAD), f32),
            jax.ShapeDtypeStruct((B, 1, NPAD), jnp.int32),
            jax.ShapeDtypeStruct((B, 1, NPAD), f32),
            jax.ShapeDtypeStruct((B, 1, NPAD), f32),
            jax.ShapeDtypeStruct((B, 1, NPAD), f32),
            jax.ShapeDtypeStruct((B, 1, NPAD), f32),
        ],
    )(ft, at, image_sizes.reshape(B, 1, 2))

    scr = sc.reshape(B, NB, BLK)
    lr = l.reshape(B, NB, BLK)
    tr = t.reshape(B, NB, BLK)
    rr = r.reshape(B, NB, BLK)
    br = b.reshape(B, NB, BLK)
    labf = lab.astype(f32).reshape(B, NB, BLK)

    row_spec = pl.BlockSpec((1, NB, BLK), lambda i: (i, 0, 0))
    smem2 = pl.BlockSpec((1, 1, 2), lambda i: (i, 0, 0), memory_space=pltpu.SMEM)
    smem1 = pl.BlockSpec((1, 1), lambda i: (0, 0), memory_space=pltpu.SMEM)
    outs = pl.pallas_call(
        _nms_body,
        grid=(B,),
        in_specs=[row_spec] * 6 + [smem2, smem2, smem1, smem1],
        out_specs=[row_spec] * 6,
        out_shape=[jax.ShapeDtypeStruct((B, NB, BLK), f32)] * 6,
        scratch_shapes=[
            pltpu.VMEM((NPAD, 8), f32),
            pltpu.VMEM((NPAD, 16), jnp.bfloat16),
            pltpu.VMEM((NPAD, 4 * BLK), jnp.bfloat16),
            pltpu.VMEM((NPAD, 8), f32),
            pltpu.VMEM((NB * 8, BLK), f32),
            pltpu.VMEM((NB, BLK), f32),
            pltpu.VMEM((NB, BLK), f32),
            pltpu.VMEM((NB, 8), f32),
            pltpu.VMEM((BLK, BLK), f32),
            pltpu.VMEM((BLK, 8), f32),
        ],
    )(scr, lr, tr, rr, br, labf, image_sizes.reshape(B, 1, 2),
      image_sizes_ori.reshape(B, 1, 2), st, nt)
    kl, kt, kr, kb, ksc, keepf = outs

    out_boxes = jnp.stack([
        kl.reshape(B, NPAD)[:, :N], kt.reshape(B, NPAD)[:, :N],
        kr.reshape(B, NPAD)[:, :N], kb.reshape(B, NPAD)[:, :N]], axis=-1)
    out_scores = ksc.reshape(B, NPAD)[:, :N]
    out_labels = lab.reshape(B, NPAD)[:, :N]
    out_keep = keepf.reshape(B, NPAD)[:, :N] > 0.5
    return (out_boxes, out_scores, out_labels, out_keep)


# speedup vs baseline: 2.2904x; 1.1256x over previous
import jax
import jax.numpy as jnp
from jax.experimental import pallas as pl
from jax.experimental.pallas import tpu as pltpu

N = 5000
NPAD = 5120
BLK = 128
NB = NPAD // BLK
NCLS = 80


def _prep_body(ft_ref, at_ref, isz_ref, sc_ref, lab_ref, l_ref, t_ref, r_ref, b_ref):
    f = ft_ref[0]
    H = isz_ref[0, 0, 0].astype(jnp.float32)
    W = isz_ref[0, 0, 1].astype(jnp.float32)
    cls = f[5:5 + NCLS, :]
    mx = jnp.max(cls, axis=0, keepdims=True)
    rowio = jax.lax.broadcasted_iota(jnp.int32, (NCLS, NPAD), 0)
    lab = jnp.min(jnp.where(cls == mx, rowio, jnp.int32(2 ** 30)), axis=0, keepdims=True)
    sc = mx * f[4:5, :]

    cell_x = at_ref[0:1, :]
    cell_y = at_ref[1:2, :]
    stride = at_ref[2:3, :]
    w_a = at_ref[3:4, :]
    h_a = at_ref[4:5, :]
    cx = (f[0:1, :] + cell_x) * stride
    cy = (f[1:2, :] + cell_y) * stride
    w = w_a * jnp.exp(f[2:3, :])
    h = h_a * jnp.exp(f[3:4, :])
    l_un = cx - w / 2.0
    t_un = cy - h / 2.0
    r_un = l_un + w
    b_un = t_un + h
    l = jnp.clip(l_un, 0.0, W - 1.0)
    r = jnp.clip(r_un, 0.0, W - 1.0)
    t = jnp.clip(t_un, 0.0, H - 1.0)
    b = jnp.clip(b_un, 0.0, H - 1.0)

    sc_ref[0] = sc
    lab_ref[0] = lab
    l_ref[0] = l
    t_ref[0] = t
    r_ref[0] = r
    b_ref[0] = b


def _nms_body(scr_ref, lr_ref, tr_ref, rr_ref, br_ref, labf_ref,
              isz_ref, iszo_ref, st_ref, nt_ref,
              kl_ref, kt_ref, kr_ref, kb_ref, ksc_ref, keep_ref,
              origC, gsrc, ohT, sortC, sortRows, rrowS, comboS, labMM, iouS, auxS):
    thr = st_ref[0, 0]
    nmsT = nt_ref[0, 0]
    H = isz_ref[0, 0, 0].astype(jnp.float32)
    scale = iszo_ref[0, 0, 0].astype(jnp.float32) / H

    sc = scr_ref[0]
    maxc = jnp.max(jnp.maximum(rr_ref[0], br_ref[0])) + 1.0
    nvalid = jnp.sum((sc >= thr).astype(jnp.int32))
    nb_used = (nvalid + (BLK - 1)) // BLK

    def _split3(x):
        hi = x.astype(jnp.bfloat16)
        r1 = x - hi.astype(jnp.float32)
        mid = r1.astype(jnp.bfloat16)
        r2 = r1 - mid.astype(jnp.float32)
        return [hi, mid, r2.astype(jnp.bfloat16)]

    def _build_cols(o, _):
        lab_row = labf_ref[0, pl.dslice(o, 1), :]
        off_row = lab_row * maxc
        sc_row = scr_ref[0, pl.dslice(o, 1), :]
        parts = (_split3(lr_ref[0, pl.dslice(o, 1), :] + off_row)
                 + _split3(tr_ref[0, pl.dslice(o, 1), :] + off_row)
                 + _split3(rr_ref[0, pl.dslice(o, 1), :] + off_row)
                 + _split3(br_ref[0, pl.dslice(o, 1), :] + off_row))
        stack16 = jnp.concatenate(
            parts + [(sc_row >= thr).astype(jnp.bfloat16),
                     lab_row.astype(jnp.bfloat16),
                     jnp.zeros((2, BLK), jnp.bfloat16)], axis=0)
        gsrc[pl.dslice(o * BLK, BLK), :] = jnp.transpose(stack16)
        combo_row = lab_row + (1.0 - (sc_row >= thr).astype(jnp.float32)) * 128.0
        comboS[pl.dslice(o, 1), :] = combo_row
        stack = jnp.concatenate([sc_row, jnp.zeros((1, BLK), jnp.float32),
                                 combo_row,
                                 jnp.zeros((5, BLK), jnp.float32)], axis=0)
        origC[pl.dslice(o * BLK, BLK), :] = jnp.transpose(stack)
        return 0

    jax.lax.fori_loop(0, NB, _build_cols, 0, unroll=False)

    tri = (jax.lax.broadcasted_iota(jnp.int32, (BLK, BLK), 1)
           < jax.lax.broadcasted_iota(jnp.int32, (BLK, BLK), 0))

    rrowS[:, :] = jnp.zeros((NB, BLK), jnp.float32)

    def _rank_blk(a, _):
        s_i = jnp.broadcast_to(origC[pl.dslice(a * BLK, BLK), 0:1], (BLK, BLK))
        c_i = jnp.broadcast_to(origC[pl.dslice(a * BLK, BLK), 2:3], (BLK, BLK))

        def _pair(bb, acc):
            s_j = scr_ref[0, pl.dslice(bb, 1), :]
            c_j = comboS[pl.dslice(bb, 1), :]
            T = ((c_j < c_i) | ((c_j == c_i) & (s_j > s_i))).astype(jnp.float32)
            colsum = jnp.sum(T, axis=0, keepdims=True)
            rrowS[pl.dslice(bb, 1), :] = rrowS[pl.dslice(bb, 1), :] + (128.0 - colsum)
            return acc + T

        acc = jax.lax.fori_loop(a + 1, NB, _pair,
                                jnp.zeros((BLK, BLK), jnp.float32))
        s_a = scr_ref[0, pl.dslice(a, 1), :]
        c_a = comboS[pl.dslice(a, 1), :]
        full_tie = (c_a == c_i) & (s_a == s_i)
        acc = acc + ((c_a < c_i) | ((c_a == c_i) & (s_a > s_i))
                     | (full_tie & tri)).astype(jnp.float32)
        total = jnp.sum(acc, axis=1, keepdims=True) + \
            jnp.transpose(rrowS[pl.dslice(a, 1), :])
        origC[pl.dslice(a * BLK, BLK), 1:2] = total
        return 0

    jax.lax.fori_loop(0, NB, _rank_blk, 0, unroll=False)

    GW = 4
    lane_f = jax.lax.broadcasted_iota(jnp.int32, (1, BLK), 1).astype(jnp.float32)
    lane_w = jax.lax.broadcasted_iota(jnp.int32, (1, GW * BLK), 1).astype(jnp.float32)

    def _gather_blk(r4, _):
        tgt = (r4 * (GW * BLK)).astype(jnp.float32) + lane_w

        def _oh(o, _2):
            rank_col = origC[pl.dslice(o * BLK, BLK), 1:2]
            ohT[pl.dslice(o * BLK, BLK), :] = (rank_col == tgt).astype(jnp.bfloat16)
            return 0

        jax.lax.fori_loop(0, NB, _oh, 0, unroll=8)
        rows16w = jax.lax.dot_general(gsrc[:, :], ohT[:, :],
                                      (((0,), (0,)), ((), ())),
                                      preferred_element_type=jnp.float32)
        for c4 in range(GW):
            rows16 = rows16w[:, c4 * BLK:(c4 + 1) * BLK]
            rr = r4 * GW + c4
            lo_row = rows16[0:1, :] + rows16[1:2, :] + rows16[2:3, :]
            to_row = rows16[3:4, :] + rows16[4:5, :] + rows16[5:6, :]
            ro_row = rows16[6:7, :] + rows16[7:8, :] + rows16[8:9, :]
            bo_row = rows16[9:10, :] + rows16[10:11, :] + rows16[11:12, :]
            v_row = rows16[12:13, :]
            lab_row = rows16[13:14, :]
            area_row = jnp.maximum(ro_row - lo_row, 0.0) * jnp.maximum(bo_row - to_row, 0.0)
            rows8 = jnp.concatenate([lo_row, to_row, ro_row, bo_row, v_row,
                                     jnp.zeros((1, BLK), jnp.float32), area_row,
                                     jnp.zeros((1, BLK), jnp.float32)], axis=0)
            sortRows[pl.dslice(rr * 8, 8), :] = rows8
            sortC[pl.dslice(rr * BLK, BLK), :] = jnp.transpose(rows8)
            labMM[pl.dslice(rr, 1), 0:1] = jnp.max(
                jnp.where(v_row > 0.5, lab_row, -1.0), axis=1, keepdims=True)
            labMM[pl.dslice(rr, 1), 1:2] = jnp.min(
                jnp.where(v_row > 0.5, lab_row, 1e9), axis=1, keepdims=True)
        return 0

    jax.lax.fori_loop(0, (nb_used + GW - 1) // GW, _gather_blk, 0, unroll=False)

    lane = jax.lax.broadcasted_iota(jnp.int32, (1, BLK), 1)

    def _nms_blk(q, _):
        cols = sortC[pl.dslice(q * BLK, BLK), :]
        v_col = cols[:, 4:5]
        l_i = jnp.broadcast_to(cols[:, 0:1], (BLK, BLK))
        t_i = jnp.broadcast_to(cols[:, 1:2], (BLK, BLK))
        r_i = jnp.broadcast_to(cols[:, 2:3], (BLK, BLK))
        b_i = jnp.broadcast_to(cols[:, 3:4], (BLK, BLK))
        v_i = jnp.broadcast_to(v_col, (BLK, BLK))
        area_i = jnp.broadcast_to(cols[:, 6:7], (BLK, BLK))

        def _iou_rows(rows8):
            l_j = rows8[0:1, :]
            t_j = rows8[1:2, :]
            r_j = rows8[2:3, :]
            b_j = rows8[3:4, :]
            area_j = rows8[6:7, :]
            iw = jnp.maximum(jnp.minimum(r_i, r_j) - jnp.maximum(l_i, l_j), 0.0)
            ih = jnp.maximum(jnp.minimum(b_i, b_j) - jnp.maximum(t_i, t_j), 0.0)
            inter = iw * ih
            union = area_i + area_j - inter
            return inter / jnp.maximum(union, 1e-9)

        def _prev(p, acc):
            rows8 = sortRows[pl.dslice(p * 8, 8), :]
            iou = _iou_rows(rows8)
            keep_j = rows8[5:6, :]
            return jnp.maximum(acc, jnp.where((keep_j > 0.5) & (iou > nmsT), 1.0, 0.0))

        mm = labMM[:, 0:1]
        minq = labMM[pl.dslice(q, 1), 1:2]
        pio = jax.lax.broadcasted_iota(jnp.int32, (NB, 1), 0)
        p0 = jnp.sum(jnp.where((mm < minq) & (pio < q), 1, 0)).astype(jnp.int32)
        p_start = jnp.where(nmsT > 0.0, p0, 0)

        sup_acc = jax.lax.fori_loop(p_start, q, _prev,
                                    jnp.zeros((BLK, BLK), jnp.float32))
        sup = jnp.max(sup_acc, axis=1, keepdims=True)

        iou_in = _iou_rows(sortRows[pl.dslice(q * 8, 8), :])
        v_j = jnp.transpose(v_col)
        conflict = jnp.max(jnp.where(tri & (iou_in > nmsT) & (v_i > 0.5) & (v_j > 0.5),
                                     1.0, 0.0)) > 0.5
        fast = v_col * (1.0 - sup)
        sortRows[pl.dslice(q * 8 + 5, 1), :] = jnp.transpose(fast)

        @pl.when(conflict)
        def _slow():
            iouS[:, :] = iou_in
            auxS[:, 0:1] = sup
            auxS[:, 1:2] = v_col

            def _seq(k, kb):
                row = iouS[pl.dslice(k, 1), :]
                sup_k = auxS[pl.dslice(k, 1), 0:1]
                val_k = auxS[pl.dslice(k, 1), 1:2]
                inblk = jnp.max(jnp.where((lane < k) & (kb > 0.5) & (row > nmsT), 1.0, 0.0),
                                axis=1, keepdims=True)
                kept = val_k * (1.0 - jnp.maximum(sup_k, inblk))
                return jnp.where(lane == k, kept, kb)

            kb = jax.lax.fori_loop(0, BLK, _seq, jnp.zeros((1, BLK), jnp.float32))
            sortRows[pl.dslice(q * 8 + 5, 1), :] = kb
        return 0

    jax.lax.fori_loop(0, nb_used, _nms_blk, 0, unroll=False)

    def _unperm(o, _):
        rank_col = jnp.broadcast_to(origC[pl.dslice(o * BLK, BLK), 1:2],
                                    (BLK, GW * BLK))

        def _scan(s4, acc):
            keep_row = jnp.concatenate(
                [sortRows[pl.dslice((s4 * GW + k) * 8 + 5, 1), :] for k in range(GW)],
                axis=1)
            srcpos = (s4 * (GW * BLK)).astype(jnp.float32) + lane_w
            hit = jnp.where((rank_col == srcpos) & (keep_row > 0.5), 1.0, 0.0)
            return jnp.maximum(acc, jnp.max(hit, axis=1, keepdims=True))

        acc = jax.lax.fori_loop(0, (nb_used + GW - 1) // GW, _scan,
                                jnp.zeros((BLK, 1), jnp.float32))
        krow = jnp.transpose(acc)
        kl_ref[0, pl.dslice(o, 1), :] = lr_ref[0, pl.dslice(o, 1), :] * scale * krow
        kt_ref[0, pl.dslice(o, 1), :] = tr_ref[0, pl.dslice(o, 1), :] * scale * krow
        kr_ref[0, pl.dslice(o, 1), :] = rr_ref[0, pl.dslice(o, 1), :] * scale * krow
        kb_ref[0, pl.dslice(o, 1), :] = br_ref[0, pl.dslice(o, 1), :] * scale * krow
        ksc_ref[0, pl.dslice(o, 1), :] = scr_ref[0, pl.dslice(o, 1), :] * krow
        keep_ref[0, pl.dslice(o, 1), :] = krow
        return 0

    jax.lax.fori_loop(0, NB, _unperm, 0, unroll=False)


def kernel(features, anchors, image_sizes, image_sizes_ori, score_thresh, nms_thresh):
    B = features.shape[0]
    ft = jnp.transpose(features, (0, 2, 1))
    ft = jnp.pad(ft, ((0, 0), (0, 3), (0, NPAD - N)))
    at = jnp.pad(jnp.transpose(anchors, (1, 0)), ((0, 3), (0, NPAD - N)))
    st = jnp.reshape(jnp.asarray(score_thresh, jnp.float32), (1, 1))
    nt = jnp.reshape(jnp.asarray(nms_thresh, jnp.float32), (1, 1))

    f32 = jnp.float32
    sc, lab, l, t, r, b = pl.pallas_call(
        _prep_body,
        grid=(B,),
        in_specs=[
            pl.BlockSpec((1, 88, NPAD), lambda i: (i, 0, 0)),
            pl.BlockSpec((8, NPAD), lambda i: (0, 0)),
            pl.BlockSpec((1, 1, 2), lambda i: (i, 0, 0), memory_space=pltpu.SMEM),
        ],
        out_specs=[pl.BlockSpec((1, 1, NPAD), lambda i: (i, 0, 0))] * 6,
        out_shape=[
            jax.ShapeDtypeStruct((B, 1, NPAD), f32),
            jax.ShapeDtypeStruct((B, 1, NPAD), jnp.int32),
            jax.ShapeDtypeStruct((B, 1, NPAD), f32),
            jax.ShapeDtypeStruct((B, 1, NPAD), f32),
            jax.ShapeDtypeStruct((B, 1, NPAD), f32),
            jax.ShapeDtypeStruct((B, 1, NPAD), f32),
        ],
    )(ft, at, image_sizes.reshape(B, 1, 2))

    scr = sc.reshape(B, NB, BLK)
    lr = l.reshape(B, NB, BLK)
    tr = t.reshape(B, NB, BLK)
    rr = r.reshape(B, NB, BLK)
    br = b.reshape(B, NB, BLK)
    labf = lab.astype(f32).reshape(B, NB, BLK)

    row_spec = pl.BlockSpec((1, NB, BLK), lambda i: (i, 0, 0))
    smem2 = pl.BlockSpec((1, 1, 2), lambda i: (i, 0, 0), memory_space=pltpu.SMEM)
    smem1 = pl.BlockSpec((1, 1), lambda i: (0, 0), memory_space=pltpu.SMEM)
    outs = pl.pallas_call(
        _nms_body,
        grid=(B,),
        in_specs=[row_spec] * 6 + [smem2, smem2, smem1, smem1],
        out_specs=[row_spec] * 6,
        out_shape=[jax.ShapeDtypeStruct((B, NB, BLK), f32)] * 6,
        scratch_shapes=[
            pltpu.VMEM((NPAD, 8), f32),
            pltpu.VMEM((NPAD, 16), jnp.bfloat16),
            pltpu.VMEM((NPAD, 4 * BLK), jnp.bfloat16),
            pltpu.VMEM((NPAD, 8), f32),
            pltpu.VMEM((NB * 8, BLK), f32),
            pltpu.VMEM((NB, BLK), f32),
            pltpu.VMEM((NB, BLK), f32),
            pltpu.VMEM((NB, 8), f32),
            pltpu.VMEM((BLK, BLK), f32),
            pltpu.VMEM((BLK, 8), f32),
        ],
    )(scr, lr, tr, rr, br, labf, image_sizes.reshape(B, 1, 2),
      image_sizes_ori.reshape(B, 1, 2), st, nt)
    kl, kt, kr, kb, ksc, keepf = outs

    out_boxes = jnp.stack([
        kl.reshape(B, NPAD)[:, :N], kt.reshape(B, NPAD)[:, :N],
        kr.reshape(B, NPAD)[:, :N], kb.reshape(B, NPAD)[:, :N]], axis=-1)
    out_scores = ksc.reshape(B, NPAD)[:, :N]
    out_labels = lab.reshape(B, NPAD)[:, :N]
    out_keep = keepf.reshape(B, NPAD)[:, :N] > 0.5
    return (out_boxes, out_scores, out_labels, out_keep)
